# baseline (speedup 1.0000x reference)
import functools

import jax
import jax.numpy as jnp
from jax import lax
from jax.experimental import pallas as pl
from jax.experimental.pallas import tpu as pltpu
from jax.experimental.pallas import tpu_sc as plsc

E = 64
B_TOK = 4096
L_TOK = 200
N_IDX = B_TOK * L_TOK
R_TAB = 100000

_info = plsc.get_sparse_core_info()
NC, NS = _info.num_cores, _info.num_subcores
NW = NC * NS
CHUNK = 128
B_PER_W = N_IDX // NW
N_CHUNKS = B_PER_W // CHUNK


def _proj_body(tab_ref, w_ref, b_ref, out_ref):
    out_ref[...] = (
        jnp.dot(tab_ref[...], w_ref[...], preferred_element_type=jnp.float32)
        + b_ref[...]
    )


def _project_table(tab2, W2, b2):
    rows = tab2.shape[0]
    blk = 2000
    grid = rows // blk
    return pl.pallas_call(
        _proj_body,
        grid=(grid,),
        in_specs=[
            pl.BlockSpec((blk, 2 * E), lambda i: (i, 0)),
            pl.BlockSpec((2 * E, 2 * E), lambda i: (0, 0)),
            pl.BlockSpec((1, 2 * E), lambda i: (0, 0)),
        ],
        out_specs=pl.BlockSpec((blk, 2 * E), lambda i: (i, 0)),
        out_shape=jax.ShapeDtypeStruct((rows, 2 * E), jnp.float32),
    )(tab2, W2, b2)


NBUF = 4
LANES = 16


def _gather_body(table_hbm, idx_hbm, out_hbm, idx_v, gbuf_v, tbuf_v, gsem, osem):
    wid = lax.axis_index("s") * NC + lax.axis_index("c")
    pltpu.sync_copy(idx_hbm.at[:, wid], idx_v)

    def out_slice(j):
        return out_hbm.at[j, :, wid]

    def start_gather(j, b):
        pltpu.async_copy(table_hbm.at[idx_v.at[j]], gbuf_v.at[b], gsem.at[b])

    def wait_gather(j, b):
        pltpu.make_async_copy(
            table_hbm.at[idx_v.at[j]], gbuf_v.at[b], gsem.at[b]
        ).wait()

    def start_wb(j, b):
        pltpu.async_copy(tbuf_v.at[b], out_slice(j), osem.at[b])

    def wait_wb(j, b):
        pltpu.make_async_copy(tbuf_v.at[b], out_slice(j), osem.at[b]).wait()

    def transpose(b):
        gb = gbuf_v.at[b]
        tb = tbuf_v.at[b]

        def tbody(e, carry):
            e8 = lax.div(e, 8)
            es = lax.rem(e, 8)
            col = jnp.full((LANES,), e, jnp.int32)
            for blg in range(CHUNK // LANES):
                rows = blg * LANES + lax.iota(jnp.int32, LANES)
                tb[e8, es, pl.ds(blg * LANES, LANES)] = plsc.load_gather(
                    gb, [rows, col]
                )
            return carry

        lax.fori_loop(0, E, tbody, 0)

    for b in range(NBUF):
        start_gather(b, b)

    def body(j, carry):
        b = lax.rem(j, NBUF)
        wait_gather(j, b)

        @pl.when(j >= NBUF)
        def _():
            wait_wb(j - NBUF, b)

        transpose(b)
        start_wb(j, b)

        @pl.when(j + NBUF < N_CHUNKS)
        def _():
            start_gather(j + NBUF, b)

        return carry

    lax.fori_loop(0, N_CHUNKS, body, 0)

    for j in range(N_CHUNKS - NBUF, N_CHUNKS):
        wait_wb(j, j % NBUF)


@functools.partial(jax.jit, static_argnums=())
def _sc_gather(table, idx5):
    mesh = plsc.VectorSubcoreMesh(core_axis_name="c", subcore_axis_name="s")
    f = pl.kernel(
        _gather_body,
        mesh=mesh,
        compiler_params=pltpu.CompilerParams(
            use_tc_tiling_on_sc=False, needs_layout_passes=False
        ),
        out_type=jax.ShapeDtypeStruct(
            (L_TOK, E // 8, NW, 8, CHUNK), jnp.float32
        ),
        scratch_types=[
            pltpu.VMEM((N_CHUNKS, CHUNK), jnp.int32),
            pltpu.VMEM((NBUF, CHUNK, E), jnp.float32),
            pltpu.VMEM((NBUF, E // 8, 8, CHUNK), jnp.float32),
            pltpu.SemaphoreType.DMA((NBUF,)),
            pltpu.SemaphoreType.DMA((NBUF,)),
        ],
    )
    return f(table, idx5)


def kernel(tokens, embed_table, W_out, b_out):
    idx5 = tokens.astype(jnp.int32).T.reshape(L_TOK, NW, CHUNK)
    tab2 = embed_table[:R_TAB].reshape(R_TAB // 2, 2 * E)
    Wt = W_out.T
    W2 = (
        jnp.zeros((2 * E, 2 * E), jnp.float32)
        .at[:E, :E].set(Wt)
        .at[E:, E:].set(Wt)
    )
    b2 = jnp.concatenate([b_out, b_out]).reshape(1, 2 * E)
    proj2 = _project_table(tab2, W2, b2)
    table = proj2.reshape(R_TAB, E)
    out5 = _sc_gather(table, idx5)
    return out5.transpose(2, 4, 0, 1, 3).reshape(B_TOK, L_TOK, 1, E)
